# Initial kernel scaffold; baseline (speedup 1.0000x reference)
import functools

import jax
import jax.numpy as jnp
from jax import lax
from jax.experimental import pallas as pl
from jax.experimental.pallas import tpu as pltpu
from jax.experimental.pallas import tpu_sc as plsc

NUM_ROWS = 16384 * 50
DIM = 32
NC, NS = 2, 16
NW = NC * NS
BPW = NUM_ROWS // NW
CHUNK = 1024
NCHUNK = BPW // CHUNK

_mesh = plsc.VectorSubcoreMesh(core_axis_name="c", subcore_axis_name="s")


@functools.partial(
    pl.kernel,
    mesh=_mesh,
    out_type=jax.ShapeDtypeStruct((NUM_ROWS, DIM), jnp.float32),
    scratch_types=[
        pltpu.VMEM((BPW,), jnp.int32),
        pltpu.VMEM((CHUNK, DIM), jnp.float32),
        pltpu.SemaphoreType.DMA,
    ],
)
def _gather_rows(idx_hbm, table_hbm, out_hbm, idx_v, rows_v, sem):
    wid = lax.axis_index("s") * NC + lax.axis_index("c")
    base = wid * BPW
    pltpu.sync_copy(idx_hbm.at[pl.ds(base, BPW)], idx_v)

    def body(i, carry):
        off = i * CHUNK
        pltpu.async_copy(
            table_hbm.at[idx_v.at[pl.ds(off, CHUNK)]], rows_v, sem
        ).wait()
        pltpu.sync_copy(rows_v, out_hbm.at[pl.ds(base + off, CHUNK)])
        return carry

    lax.fori_loop(0, NCHUNK, body, 0)


def kernel(indices, table):
    flat = indices.reshape(-1).astype(jnp.int32)
    out = _gather_rows(flat, table)
    return out.reshape(indices.shape + (table.shape[1],))


# speedup vs baseline: 1.1034x; 1.1034x over previous
import functools

import jax
import jax.numpy as jnp
from jax import lax
from jax.experimental import pallas as pl
from jax.experimental.pallas import tpu as pltpu
from jax.experimental.pallas import tpu_sc as plsc

NUM_ROWS = 16384 * 50
DIM = 32
NC, NS = 2, 16
NW = NC * NS
BPW = NUM_ROWS // NW
CHUNK = 1024
NCHUNK = BPW // CHUNK

_mesh = plsc.VectorSubcoreMesh(core_axis_name="c", subcore_axis_name="s")


@functools.partial(
    pl.kernel,
    mesh=_mesh,
    out_type=jax.ShapeDtypeStruct((NUM_ROWS, DIM), jnp.float32),
    scratch_types=[
        pltpu.VMEM((BPW,), jnp.int32),
        pltpu.VMEM((CHUNK, DIM), jnp.float32),
        pltpu.SemaphoreType.DMA,
    ],
    compiler_params=pltpu.CompilerParams(use_tc_tiling_on_sc=False),
)
def _gather_rows(idx_hbm, table_hbm, out_hbm, idx_v, rows_v, sem):
    wid = lax.axis_index("s") * NC + lax.axis_index("c")
    base = wid * BPW
    pltpu.sync_copy(idx_hbm.at[pl.ds(base, BPW)], idx_v)

    def body(i, carry):
        off = i * CHUNK
        pltpu.async_copy(
            table_hbm.at[idx_v.at[pl.ds(off, CHUNK)]], rows_v, sem
        ).wait()
        pltpu.sync_copy(rows_v, out_hbm.at[pl.ds(base + off, CHUNK)])
        return carry

    lax.fori_loop(0, NCHUNK, body, 0)


def kernel(indices, table):
    flat = indices.reshape(-1).astype(jnp.int32)
    out = _gather_rows(flat, table)
    return out.reshape(indices.shape + (table.shape[1],))


# speedup vs baseline: 1.1142x; 1.0098x over previous
import functools

import jax
import jax.numpy as jnp
from jax import lax
from jax.experimental import pallas as pl
from jax.experimental.pallas import tpu as pltpu
from jax.experimental.pallas import tpu_sc as plsc

NUM_ROWS = 16384 * 50
DIM = 32
NC, NS = 2, 16
NW = NC * NS
BPW = NUM_ROWS // NW
NBUF = 4
CHUNK = 640
NCHUNK = BPW // CHUNK

_mesh = plsc.VectorSubcoreMesh(core_axis_name="c", subcore_axis_name="s")


@functools.partial(
    pl.kernel,
    mesh=_mesh,
    out_type=jax.ShapeDtypeStruct((NUM_ROWS, DIM), jnp.float32),
    scratch_types=[
        pltpu.VMEM((BPW,), jnp.int32),
        [pltpu.VMEM((CHUNK, DIM), jnp.float32) for _ in range(NBUF)],
        [pltpu.SemaphoreType.DMA for _ in range(NBUF)],
        [pltpu.SemaphoreType.DMA for _ in range(NBUF)],
    ],
    compiler_params=pltpu.CompilerParams(use_tc_tiling_on_sc=False),
)
def _gather_rows(idx_hbm, table_hbm, out_hbm, idx_v, rows, sem_in, sem_out):
    wid = lax.axis_index("s") * NC + lax.axis_index("c")
    base = wid * BPW
    pltpu.sync_copy(idx_hbm.at[pl.ds(base, BPW)], idx_v)

    def in_copy(i, b):
        return pltpu.make_async_copy(
            table_hbm.at[idx_v.at[pl.ds(i * CHUNK, CHUNK)]], rows[b], sem_in[b]
        )

    def out_copy(i, b):
        return pltpu.make_async_copy(
            rows[b], out_hbm.at[pl.ds(base + i * CHUNK, CHUNK)], sem_out[b]
        )

    for b in range(NBUF):
        in_copy(b, b).start()

    @pl.loop(0, NCHUNK - NBUF, step=NBUF)
    def _group(g):
        for b in range(NBUF):
            i = g + b
            in_copy(i, b).wait()
            out_copy(i, b).start()
            out_copy(i, b).wait()
            in_copy(i + NBUF, b).start()

    for b in range(NBUF):
        in_copy(NCHUNK - NBUF + b, b).wait()
        out_copy(NCHUNK - NBUF + b, b).start()
    for b in range(NBUF):
        out_copy(NCHUNK - NBUF + b, b).wait()


def kernel(indices, table):
    flat = indices.reshape(-1).astype(jnp.int32)
    out = _gather_rows(flat, table)
    return out.reshape(indices.shape + (table.shape[1],))


# speedup vs baseline: 1.2818x; 1.1504x over previous
import functools

import jax
import jax.numpy as jnp
from jax import lax
from jax.experimental import pallas as pl
from jax.experimental.pallas import tpu as pltpu
from jax.experimental.pallas import tpu_sc as plsc

NUM_TABLE_ROWS = 1000000
DIM = 32
B_SAMPLES = 16384
SLOTS = 50
NC, NS = 2, 16
NW = NC * NS
NTJ = B_SAMPLES // 128
TJ_PER_W = NTJ // NW
UNITS = SLOTS * TJ_PER_W

_mesh = plsc.VectorSubcoreMesh(core_axis_name="c", subcore_axis_name="s")


@functools.partial(
    pl.kernel,
    mesh=_mesh,
    out_type=jax.ShapeDtypeStruct((SLOTS, 4 * NTJ, 8 * 128), jnp.float32),
    scratch_types=[
        pltpu.VMEM((128,), jnp.int32),
        pltpu.VMEM((128, DIM), jnp.float32),
        pltpu.VMEM((DIM * 128,), jnp.float32),
        pltpu.SemaphoreType.DMA,
    ],
    compiler_params=pltpu.CompilerParams(
        use_tc_tiling_on_sc=False, needs_layout_passes=False
    ),
)
def _lookup(idx_hbm, table_hbm, out_hbm, idx_v, rows_v, tbuf_v, sem):
    wid = lax.axis_index("s") * NC + lax.axis_index("c")
    tj0 = wid * TJ_PER_W
    iota = lax.iota(jnp.int32, 16)

    @pl.loop(0, UNITS)
    def _unit(u):
        s = jnp.right_shift(u, 2)
        tj = tj0 + jnp.bitwise_and(u, 3)
        pltpu.sync_copy(idx_hbm.at[s, pl.ds(tj * 128, 128)], idx_v)
        pltpu.async_copy(table_hbm.at[idx_v], rows_v, sem).wait()
        for v in range(256):
            p0 = 16 * v
            row_vec = iota + (p0 % 128)
            col_vec = jnp.full((16,), p0 // 128, jnp.int32)
            tbuf_v[pl.ds(p0, 16)] = plsc.load_gather(rows_v, [row_vec, col_vec])
        for ti in range(4):
            pltpu.sync_copy(
                tbuf_v.at[pl.ds(ti * 1024, 1024)],
                out_hbm.at[s, ti * 128 + tj],
            )


def kernel(indices, table):
    out3d = _lookup(indices.T.astype(jnp.int32), table)
    o = out3d.reshape(SLOTS, 4, 128, 8, 128)
    o = o.transpose(0, 1, 3, 2, 4)
    o = o.reshape(SLOTS, DIM, B_SAMPLES)
    return o.transpose(2, 0, 1)


# speedup vs baseline: 1.5352x; 1.1977x over previous
import functools

import jax
import jax.numpy as jnp
from jax import lax
from jax.experimental import pallas as pl
from jax.experimental.pallas import tpu as pltpu
from jax.experimental.pallas import tpu_sc as plsc

NUM_TABLE_ROWS = 1000000
DIM = 32
B_SAMPLES = 16384
SLOTS = 50
NC, NS = 2, 16
NW = NC * NS
NTJ = B_SAMPLES // 128
TJ_PER_W = NTJ // NW
UNITS = SLOTS * TJ_PER_W

_mesh = plsc.VectorSubcoreMesh(core_axis_name="c", subcore_axis_name="s")


@functools.partial(
    pl.kernel,
    mesh=_mesh,
    out_type=jax.ShapeDtypeStruct((SLOTS, 4 * NTJ, 8 * 128), jnp.float32),
    scratch_types=[
        pltpu.VMEM((SLOTS, 128 * TJ_PER_W), jnp.int32),
        [pltpu.VMEM((128, DIM), jnp.float32) for _ in range(2)],
        [pltpu.VMEM((4, 1024), jnp.float32) for _ in range(2)],
        [pltpu.SemaphoreType.DMA for _ in range(2)],
        [pltpu.SemaphoreType.DMA for _ in range(2)],
    ],
    compiler_params=pltpu.CompilerParams(
        use_tc_tiling_on_sc=False, needs_layout_passes=False
    ),
)
def _lookup(idx_hbm, table_hbm, out_hbm, idx_v, rows, tbufs, sem_in, sem_out):
    wid = lax.axis_index("s") * NC + lax.axis_index("c")
    tj0 = wid * TJ_PER_W
    iota = lax.iota(jnp.int32, 16)

    pltpu.sync_copy(idx_hbm.at[:, pl.ds(tj0 * 128, 128 * TJ_PER_W)], idx_v)

    def in_copy(u, b):
        s = jnp.right_shift(u, 2)
        j = jnp.bitwise_and(u, 3)
        return pltpu.make_async_copy(
            table_hbm.at[idx_v.at[s, pl.ds(j * 128, 128)]], rows[b], sem_in[b]
        )

    def out_copy(u, b, ti):
        s = jnp.right_shift(u, 2)
        tj = tj0 + jnp.bitwise_and(u, 3)
        return pltpu.make_async_copy(
            tbufs[b].at[ti], out_hbm.at[s, ti * 128 + tj], sem_out[b]
        )

    def drain_outs(b):
        pltpu.make_async_copy(
            out_hbm.at[0, pl.ds(0, 4)], tbufs[b], sem_out[b]
        ).wait()

    def transpose(b):
        for v in range(256):
            row_vec = iota + (16 * v % 128)
            col_vec = jnp.full((16,), v // 8, jnp.int32)
            tbufs[b][v // 64, pl.ds(16 * v % 1024, 16)] = plsc.load_gather(
                rows[b], [row_vec, col_vec]
            )

    in_copy(0, 0).start()

    @pl.loop(0, UNITS // 2 - 1)
    def _pair(g):
        for b in range(2):
            u = 2 * g + b
            in_copy(u, b).wait()
            in_copy(u + 1, 1 - b).start()
            pl.when(g > 0)(lambda b=b: drain_outs(b))
            transpose(b)
            for ti in range(4):
                out_copy(u, b, ti).start()

    for b in range(2):
        u = UNITS - 2 + b
        in_copy(u, b).wait()
        if b == 0:
            in_copy(u + 1, 1).start()
        drain_outs(b)
        transpose(b)
        for ti in range(4):
            out_copy(u, b, ti).start()
    for b in range(2):
        drain_outs(b)


def kernel(indices, table):
    out3d = _lookup(indices.T.astype(jnp.int32), table)
    o = out3d.reshape(SLOTS, 4, 128, 8, 128)
    o = o.transpose(0, 1, 3, 2, 4)
    o = o.reshape(SLOTS, DIM, B_SAMPLES)
    return o.transpose(2, 0, 1)


# speedup vs baseline: 1.9208x; 1.2512x over previous
import functools

import jax
import jax.numpy as jnp
from jax import lax
from jax.experimental import pallas as pl
from jax.experimental.pallas import tpu as pltpu
from jax.experimental.pallas import tpu_sc as plsc

NUM_TABLE_ROWS = 1000000
DIM = 32
B_SAMPLES = 16384
SLOTS = 50
NC, NS = 2, 16
NW = NC * NS
NTJ = B_SAMPLES // 128
TJ_PER_W = NTJ // NW
UNITS = SLOTS * TJ_PER_W

_mesh = plsc.VectorSubcoreMesh(core_axis_name="c", subcore_axis_name="s")


@functools.partial(
    pl.kernel,
    mesh=_mesh,
    out_type=jax.ShapeDtypeStruct((SLOTS, 4 * NTJ, 8 * 128), jnp.float32),
    scratch_types=[
        pltpu.VMEM((SLOTS, 128 * TJ_PER_W), jnp.int32),
        [pltpu.VMEM((128, DIM), jnp.float32) for _ in range(2)],
        [pltpu.VMEM((4, 1024), jnp.float32) for _ in range(2)],
        [pltpu.SemaphoreType.DMA for _ in range(2)],
        [pltpu.SemaphoreType.DMA for _ in range(2)],
    ],
    compiler_params=pltpu.CompilerParams(
        use_tc_tiling_on_sc=False, needs_layout_passes=False
    ),
)
def _lookup(idx_hbm, table_hbm, out_hbm, idx_v, rows, tbufs, sem_in, sem_out):
    wid = lax.axis_index("s") * NC + lax.axis_index("c")
    tj0 = wid * TJ_PER_W
    iota = lax.iota(jnp.int32, 16)

    pltpu.sync_copy(idx_hbm.at[:, pl.ds(tj0 * 128, 128 * TJ_PER_W)], idx_v)

    def in_copy(u, b):
        s = jnp.right_shift(u, 2)
        j = jnp.bitwise_and(u, 3)
        return pltpu.make_async_copy(
            table_hbm.at[idx_v.at[s, pl.ds(j * 128, 128)]], rows[b], sem_in[b]
        )

    def out_copy(u, b, ti):
        s = jnp.right_shift(u, 2)
        tj = tj0 + jnp.bitwise_and(u, 3)
        return pltpu.make_async_copy(
            tbufs[b].at[ti], out_hbm.at[s, ti * 128 + tj], sem_out[b]
        )

    def drain_outs(b):
        pltpu.make_async_copy(
            out_hbm.at[0, pl.ds(0, 4)], tbufs[b], sem_out[b]
        ).wait()

    def transpose(b):
        @plsc.parallel_loop(0, 256, unroll=8)
        def _t(v):
            p0 = v * 16
            row_vec = iota + jnp.bitwise_and(p0, 127)
            col_vec = jnp.zeros((16,), jnp.int32) + jnp.right_shift(v, 3)
            ti = jnp.right_shift(v, 6)
            off = pl.multiple_of(jnp.bitwise_and(p0, 1023), 16)
            tbufs[b][ti, pl.ds(off, 16)] = plsc.load_gather(
                rows[b], [row_vec, col_vec]
            )

    in_copy(0, 0).start()

    @pl.loop(0, UNITS // 2 - 1)
    def _pair(g):
        for b in range(2):
            u = 2 * g + b
            in_copy(u, b).wait()
            in_copy(u + 1, 1 - b).start()
            pl.when(g > 0)(lambda b=b: drain_outs(b))
            transpose(b)
            for ti in range(4):
                out_copy(u, b, ti).start()

    for b in range(2):
        u = UNITS - 2 + b
        in_copy(u, b).wait()
        if b == 0:
            in_copy(u + 1, 1).start()
        drain_outs(b)
        transpose(b)
        for ti in range(4):
            out_copy(u, b, ti).start()
    for b in range(2):
        drain_outs(b)


def kernel(indices, table):
    out3d = _lookup(indices.T.astype(jnp.int32), table)
    o = out3d.reshape(SLOTS, 4, 128, 8, 128)
    o = o.transpose(0, 1, 3, 2, 4)
    o = o.reshape(SLOTS, DIM, B_SAMPLES)
    return o.transpose(2, 0, 1)
